# Initial kernel scaffold; baseline (speedup 1.0000x reference)
import jax
import jax.numpy as jnp
from jax.experimental import pallas as pl


def kernel(hidden_states, W, input_ids, seg_token_mask, num_patches):
    raise NotImplementedError("write your pallas kernel here")


# speedup vs baseline: 7.0095x; 7.0095x over previous
import jax
import jax.numpy as jnp
from jax import lax
from jax.experimental import pallas as pl
from jax.experimental.pallas import tpu as pltpu
from jax.experimental.pallas import tpu_sc as plsc

_IMAGE_TOKEN_INDEX = -200

_L, _B, _T, _D = 33, 2, 2048, 768
_P = 576
_N = 3 * _B
_LP = 48
_RPW = _P // 16


def _sc_body(hs_ref, w_ref, meta_ref, g_ref, out1_ref, out2_ref,
             meta_v, idx_v, seg_v, w_v, g_v, pub_v, base_v, shared, stage_v,
             sem):
    c = lax.axis_index("c")
    s = lax.axis_index("s")

    i16 = lax.iota(jnp.int32, 16)

    @pl.when(s < 3)
    def _compute():
        n = c * 3 + s
        pltpu.sync_copy(meta_ref, meta_v)
        m = meta_v[...]
        img = jnp.sum(jnp.where(i16 == c, m, 0))
        col = jnp.sum(jnp.where(i16 == _B + n, m, 0))
        for k in range(_LP // 16):
            lv = jnp.minimum(i16 + 16 * k, _L - 1)
            idx_v[pl.ds(16 * k, 16)] = lv * (_B * _T) + c * _T + col
        pltpu.async_copy(hs_ref.at[idx_v], seg_v, sem).wait()
        pltpu.sync_copy(w_ref, w_v)
        pltpu.sync_copy(g_ref.at[n], g_v)

        def lbody(l, carry):
            lg0, lg1, lg2 = carry
            acc = seg_v[l, pl.ds(0, 16)] * w_v[l, pl.ds(0, 16)]
            for k in range(1, _D // 16):
                acc = acc + seg_v[l, pl.ds(16 * k, 16)] * w_v[l, pl.ds(16 * k, 16)]
            tot = jnp.sum(acc)
            return (jnp.where(i16 == l, tot, lg0),
                    jnp.where(i16 + 16 == l, tot, lg1),
                    jnp.where(i16 + 32 == l, tot, lg2))

        ninf = jnp.full((16,), -jnp.inf, jnp.float32)
        lg0, lg1, lg2 = lax.fori_loop(0, _L, lbody, (ninf, ninf, ninf))

        v0 = lg0 + g_v[pl.ds(0, 16)]
        v1 = lg1 + g_v[pl.ds(16, 16)]
        v2 = lg2 + g_v[pl.ds(32, 16)]
        mx = jnp.maximum(jnp.maximum(jnp.max(v0), jnp.max(v1)), jnp.max(v2))
        big = jnp.int32(1 << 20)
        barg = jnp.minimum(
            jnp.minimum(jnp.min(jnp.where(v0 >= mx, i16, big)),
                        jnp.min(jnp.where(v1 >= mx, i16 + 16, big))),
            jnp.min(jnp.where(v2 >= mx, i16 + 32, big)))

        pltpu.sync_copy(seg_v.at[pl.ds(barg, 1)], out1_ref.at[pl.ds(n, 1)])
        rb = barg * (_B * _T) + c * _T + img
        pub_v[0] = jnp.zeros((16,), jnp.int32) + rb
        pltpu.sync_copy(pub_v, shared.at[pl.ds(s, 1)])

    plsc.subcore_barrier()

    pltpu.sync_copy(shared, base_v)
    for j in range(3):
        rb = base_v[j][0]
        src0 = rb + s * _RPW
        dst0 = (c * 3 + j) * _P + s * _RPW
        pltpu.sync_copy(hs_ref.at[pl.ds(src0, _RPW)], stage_v)
        pltpu.sync_copy(stage_v, out2_ref.at[pl.ds(dst0, _RPW)])


def kernel(hidden_states, W, input_ids, seg_token_mask, num_patches):
    del num_patches
    L, B, T, D = hidden_states.shape
    hs_flat = hidden_states.reshape(L * B * T, D)

    img_idx = jnp.argmax(input_ids == _IMAGE_TOKEN_INDEX, axis=1)
    _, cols = jnp.nonzero(seg_token_mask, size=_N)
    meta = jnp.concatenate([
        img_idx.astype(jnp.int32),
        cols.astype(jnp.int32),
        jnp.zeros((16 - _B - _N,), jnp.int32),
    ])
    g = jax.random.gumbel(jax.random.key(1), (_N, _L), jnp.float32)
    g_pad = jnp.zeros((_N, _LP), jnp.float32).at[:, :_L].set(g)

    mesh = plsc.VectorSubcoreMesh(core_axis_name="c", subcore_axis_name="s")
    out1, out2f = pl.kernel(
        _sc_body,
        out_type=(
            jax.ShapeDtypeStruct((_N, _D), jnp.float32),
            jax.ShapeDtypeStruct((_N * _P, _D), jnp.float32),
        ),
        mesh=mesh,
        compiler_params=pltpu.CompilerParams(
            use_tc_tiling_on_sc=False, needs_layout_passes=False),
        scratch_types=[
            pltpu.VMEM((16,), jnp.int32),
            pltpu.VMEM((_LP,), jnp.int32),
            pltpu.VMEM((_LP, _D), jnp.float32),
            pltpu.VMEM((_L, _D), jnp.float32),
            pltpu.VMEM((_LP,), jnp.float32),
            pltpu.VMEM((1, 16), jnp.int32),
            pltpu.VMEM((3, 16), jnp.int32),
            pltpu.VMEM_SHARED((3, 16), jnp.int32),
            pltpu.VMEM((_RPW, _D), jnp.float32),
            pltpu.SemaphoreType.DMA,
        ],
    )(hs_flat, W, meta, g_pad)

    out2 = out2f.reshape(_N, _P, _D)
    return (out1, out2, out1)


# speedup vs baseline: 7.0666x; 1.0081x over previous
import jax
import jax.numpy as jnp
from jax import lax
from jax.experimental import pallas as pl
from jax.experimental.pallas import tpu as pltpu
from jax.experimental.pallas import tpu_sc as plsc

_IMAGE_TOKEN_INDEX = -200

_L, _B, _T, _D = 33, 2, 2048, 768
_P = 576
_N = 3 * _B
_LP = 48
_RPW = _P // 16


def _sc_body(hs_ref, w_ref, meta_ref, g_ref, out1_ref, out2_ref,
             meta_v, idx_v, seg_v, w_v, g_v, pub_v, base_v, shared, stage_v,
             sem):
    c = lax.axis_index("c")
    s = lax.axis_index("s")

    i16 = lax.iota(jnp.int32, 16)

    @pl.when(s < 3)
    def _compute():
        n = c * 3 + s
        pltpu.sync_copy(meta_ref, meta_v)
        m = meta_v[...]
        img = jnp.sum(jnp.where(i16 == c, m, 0))
        col = jnp.sum(jnp.where(i16 == _B + n, m, 0))
        for k in range(_LP // 16):
            lv = jnp.minimum(i16 + 16 * k, _L - 1)
            idx_v[pl.ds(16 * k, 16)] = lv * (_B * _T) + c * _T + col
        pltpu.async_copy(hs_ref.at[idx_v], seg_v, sem).wait()
        pltpu.sync_copy(w_ref, w_v)
        pltpu.sync_copy(g_ref.at[n], g_v)

        def lbody(l, carry):
            lg0, lg1, lg2 = carry
            acc = seg_v[l, pl.ds(0, 16)] * w_v[l, pl.ds(0, 16)]
            for k in range(1, _D // 16):
                acc = acc + seg_v[l, pl.ds(16 * k, 16)] * w_v[l, pl.ds(16 * k, 16)]
            tot = jnp.sum(acc)
            return (jnp.where(i16 == l, tot, lg0),
                    jnp.where(i16 + 16 == l, tot, lg1),
                    jnp.where(i16 + 32 == l, tot, lg2))

        ninf = jnp.full((16,), -jnp.inf, jnp.float32)
        lg0, lg1, lg2 = lax.fori_loop(0, _L, lbody, (ninf, ninf, ninf))

        v0 = lg0 + g_v[pl.ds(0, 16)]
        v1 = lg1 + g_v[pl.ds(16, 16)]
        v2 = lg2 + g_v[pl.ds(32, 16)]
        mx = jnp.maximum(jnp.maximum(jnp.max(v0), jnp.max(v1)), jnp.max(v2))
        big = jnp.int32(1 << 20)
        barg = jnp.minimum(
            jnp.minimum(jnp.min(jnp.where(v0 >= mx, i16, big)),
                        jnp.min(jnp.where(v1 >= mx, i16 + 16, big))),
            jnp.min(jnp.where(v2 >= mx, i16 + 32, big)))

        pltpu.sync_copy(seg_v.at[pl.ds(barg, 1)], out1_ref.at[pl.ds(n, 1)])
        rb = barg * (_B * _T) + c * _T + img
        pub_v[0] = jnp.zeros((16,), jnp.int32) + rb
        pltpu.sync_copy(pub_v, shared.at[pl.ds(s, 1)])

    plsc.subcore_barrier()

    pltpu.sync_copy(shared, base_v)
    for j in range(3):
        rb = base_v[j][0]
        src0 = rb + s * _RPW
        dst0 = (c * 3 + j) * _P + s * _RPW
        pltpu.sync_copy(hs_ref.at[pl.ds(src0, _RPW)], stage_v)
        pltpu.sync_copy(stage_v, out2_ref.at[pl.ds(dst0, _RPW)])


def kernel(hidden_states, W, input_ids, seg_token_mask, num_patches):
    del num_patches
    L, B, T, D = hidden_states.shape
    hs_flat = hidden_states.reshape(L * B * T, D)

    it = jnp.arange(T, dtype=jnp.int32)[None, :]
    img_idx = jnp.min(
        jnp.where(input_ids == _IMAGE_TOKEN_INDEX, it, T), axis=1)
    c0 = jnp.min(jnp.where(seg_token_mask, it, T), axis=1)
    c1 = jnp.min(jnp.where(seg_token_mask & (it > c0[:, None]), it, T), axis=1)
    c2 = jnp.min(jnp.where(seg_token_mask & (it > c1[:, None]), it, T), axis=1)
    cols = jnp.stack([c0, c1, c2], axis=1).reshape(_N)
    meta = jnp.concatenate([
        img_idx.astype(jnp.int32),
        cols.astype(jnp.int32),
        jnp.zeros((16 - _B - _N,), jnp.int32),
    ])
    g = jax.random.gumbel(jax.random.key(1), (_N, _L), jnp.float32)
    g_pad = jnp.concatenate(
        [g, jnp.zeros((_N, _LP - _L), jnp.float32)], axis=1)

    mesh = plsc.VectorSubcoreMesh(core_axis_name="c", subcore_axis_name="s")
    out1, out2f = pl.kernel(
        _sc_body,
        out_type=(
            jax.ShapeDtypeStruct((_N, _D), jnp.float32),
            jax.ShapeDtypeStruct((_N * _P, _D), jnp.float32),
        ),
        mesh=mesh,
        compiler_params=pltpu.CompilerParams(
            use_tc_tiling_on_sc=False, needs_layout_passes=False),
        scratch_types=[
            pltpu.VMEM((16,), jnp.int32),
            pltpu.VMEM((_LP,), jnp.int32),
            pltpu.VMEM((_LP, _D), jnp.float32),
            pltpu.VMEM((_L, _D), jnp.float32),
            pltpu.VMEM((_LP,), jnp.float32),
            pltpu.VMEM((1, 16), jnp.int32),
            pltpu.VMEM((3, 16), jnp.int32),
            pltpu.VMEM_SHARED((3, 16), jnp.int32),
            pltpu.VMEM((_RPW, _D), jnp.float32),
            pltpu.SemaphoreType.DMA,
        ],
    )(hs_flat, W, meta, g_pad)

    out2 = out2f.reshape(_N, _P, _D)
    return (out1, out2, out1)


# speedup vs baseline: 69.3485x; 9.8136x over previous
import jax
import jax.numpy as jnp
from jax import lax
from jax.experimental import pallas as pl
from jax.experimental.pallas import tpu as pltpu
from jax.experimental.pallas import tpu_sc as plsc

_IMAGE_TOKEN_INDEX = -200

_L, _B, _T, _D = 33, 2, 2048, 768
_P = 576
_N = 3 * _B
_LP = 48
_NS_C = 12
_RPW = _P // _NS_C


def _sc_body(hs_ref, w_ref, meta_ref, g_ref, out1_ref, out2_ref,
             meta_v, idx_v, seg_v, w_v, g_v, out1_v, pub_v, base_v, shared,
             stage_v, sem):
    c = lax.axis_index("c")
    s = lax.axis_index("s")
    i16 = lax.iota(jnp.int32, 16)

    @pl.when(s < 3)
    def _compute():
        n = c * 3 + s
        pltpu.sync_copy(meta_ref, meta_v)
        m = meta_v[...]
        img = jnp.sum(jnp.where(i16 == c, m, 0))
        col = jnp.sum(jnp.where(i16 == _B + n, m, 0))
        for k in range(_LP // 16):
            lv = jnp.minimum(i16 + 16 * k, _L - 1)
            idx_v[pl.ds(16 * k, 16)] = lv * (_B * _T) + c * _T + col
        pltpu.async_copy(hs_ref.at[idx_v], seg_v, sem).wait()
        pltpu.sync_copy(w_ref, w_v)
        pltpu.sync_copy(g_ref.at[pl.ds(n * _LP, _LP)], g_v)

        def lbody(l, carry):
            lg0, lg1, lg2 = carry
            acc = seg_v[l, pl.ds(0, 16)] * w_v[l, pl.ds(0, 16)]
            for k in range(1, _D // 16):
                acc = acc + seg_v[l, pl.ds(16 * k, 16)] * w_v[l, pl.ds(16 * k, 16)]
            tot = jnp.sum(acc)
            return (jnp.where(i16 == l, tot, lg0),
                    jnp.where(i16 + 16 == l, tot, lg1),
                    jnp.where(i16 + 32 == l, tot, lg2))

        ninf = jnp.full((16,), -jnp.inf, jnp.float32)
        lg0, lg1, lg2 = lax.fori_loop(0, _L, lbody, (ninf, ninf, ninf))

        v0 = lg0 + g_v[pl.ds(0, 16)]
        v1 = lg1 + g_v[pl.ds(16, 16)]
        v2 = lg2 + g_v[pl.ds(32, 16)]
        mx = jnp.maximum(jnp.maximum(jnp.max(v0), jnp.max(v1)), jnp.max(v2))
        big = jnp.int32(1 << 20)
        barg = jnp.minimum(
            jnp.minimum(jnp.min(jnp.where(v0 >= mx, i16, big)),
                        jnp.min(jnp.where(v1 >= mx, i16 + 16, big))),
            jnp.min(jnp.where(v2 >= mx, i16 + 32, big)))

        for k in range(_D // 16):
            out1_v[pl.ds(16 * k, 16)] = seg_v[barg, pl.ds(16 * k, 16)]
        pltpu.sync_copy(out1_v, out1_ref.at[pl.ds(n * _D, _D)])
        rb = barg * (_B * _T) + c * _T + img
        pub_v[...] = jnp.zeros((16,), jnp.int32) + rb
        pltpu.sync_copy(pub_v, shared.at[pl.ds(16 * s, 16)])

    plsc.subcore_barrier()

    @pl.when(s < _NS_C)
    def _spans():
        pltpu.sync_copy(shared, base_v)
        for j in range(3):
            rb = base_v[pl.ds(16 * j, 16)][0]
            for k in range(_RPW // 16):
                idx_v[pl.ds(16 * k, 16)] = rb + s * _RPW + i16 + 16 * k
            pltpu.async_copy(hs_ref.at[idx_v], stage_v, sem).wait()
            dst0 = (c * 3 + j) * _P + s * _RPW
            pltpu.sync_copy(stage_v, out2_ref.at[pl.ds(dst0, _RPW)])


def kernel(hidden_states, W, input_ids, seg_token_mask, num_patches):
    del num_patches
    L, B, T, D = hidden_states.shape
    hs_flat = hidden_states.reshape(L * B * T, D)

    it = jnp.arange(T, dtype=jnp.int32)[None, :]
    img_idx = jnp.min(
        jnp.where(input_ids == _IMAGE_TOKEN_INDEX, it, T), axis=1)
    c0 = jnp.min(jnp.where(seg_token_mask, it, T), axis=1)
    c1 = jnp.min(jnp.where(seg_token_mask & (it > c0[:, None]), it, T), axis=1)
    c2 = jnp.min(jnp.where(seg_token_mask & (it > c1[:, None]), it, T), axis=1)
    cols = jnp.stack([c0, c1, c2], axis=1).reshape(_N)
    meta = jnp.concatenate([
        img_idx.astype(jnp.int32),
        cols.astype(jnp.int32),
        jnp.zeros((16 - _B - _N,), jnp.int32),
    ])
    g = jax.random.gumbel(jax.random.key(1), (_N, _L), jnp.float32)
    g_pad = jnp.concatenate(
        [g, jnp.zeros((_N, _LP - _L), jnp.float32)], axis=1).reshape(_N * _LP)

    mesh = plsc.VectorSubcoreMesh(core_axis_name="c", subcore_axis_name="s")
    out1f, out2f = pl.kernel(
        _sc_body,
        out_type=(
            jax.ShapeDtypeStruct((_N * _D,), jnp.float32),
            jax.ShapeDtypeStruct((_N * _P, _D), jnp.float32),
        ),
        mesh=mesh,
        compiler_params=pltpu.CompilerParams(needs_layout_passes=False),
        scratch_types=[
            pltpu.VMEM((16,), jnp.int32),
            pltpu.VMEM((_LP,), jnp.int32),
            pltpu.VMEM((_LP, _D), jnp.float32),
            pltpu.VMEM((_L, _D), jnp.float32),
            pltpu.VMEM((_LP,), jnp.float32),
            pltpu.VMEM((_D,), jnp.float32),
            pltpu.VMEM((16,), jnp.int32),
            pltpu.VMEM((_LP,), jnp.int32),
            pltpu.VMEM_SHARED((_LP,), jnp.int32),
            pltpu.VMEM((_RPW, _D), jnp.float32),
            pltpu.SemaphoreType.DMA,
        ],
    )(hs_flat, W, meta, g_pad)

    out1 = out1f.reshape(_N, _D)
    out2 = out2f.reshape(_N, _P, _D)
    return (out1, out2, out1)


# speedup vs baseline: 72.8723x; 1.0508x over previous
import jax
import jax.numpy as jnp
from jax import lax
from jax.experimental import pallas as pl
from jax.experimental.pallas import tpu as pltpu
from jax.experimental.pallas import tpu_sc as plsc

_IMAGE_TOKEN_INDEX = -200

_L, _B, _T, _D = 33, 2, 2048, 768
_P = 576
_N = 3 * _B
_LP = 48
_NS_C = 12
_RPW = _P // _NS_C


def _sc_body(hs_ref, w_ref, meta_ref, g_ref, out1_ref, out2_ref,
             meta_v, idx_v, idx2_v, seg_v, w_v, g_v, out1_v, pub_v, base_v,
             shared, stage_v, sem, sem2, sem3, sem4):
    c = lax.axis_index("c")
    s = lax.axis_index("s")
    i16 = lax.iota(jnp.int32, 16)

    @pl.when(s < 3)
    def _compute():
        n = c * 3 + s
        pltpu.sync_copy(meta_ref, meta_v)
        m = meta_v[...]
        img = jnp.sum(jnp.where(i16 == c, m, 0))
        col = jnp.sum(jnp.where(i16 == _B + n, m, 0))
        for k in range(_LP // 16):
            lv = jnp.minimum(i16 + 16 * k, _L - 1)
            idx_v[pl.ds(16 * k, 16)] = lv * (_B * _T) + c * _T + col
        gch = pltpu.async_copy(hs_ref.at[idx_v], seg_v, sem)
        wch = pltpu.async_copy(w_ref, w_v, sem2)
        pltpu.sync_copy(g_ref.at[pl.ds(n * _LP, _LP)], g_v)
        gch.wait()
        wch.wait()

        def lbody(l, carry):
            lg0, lg1, lg2 = carry
            acc = seg_v[l, pl.ds(0, 16)] * w_v[l, pl.ds(0, 16)]
            for k in range(1, _D // 16):
                acc = acc + seg_v[l, pl.ds(16 * k, 16)] * w_v[l, pl.ds(16 * k, 16)]
            tot = jnp.sum(acc)
            return (jnp.where(i16 == l, tot, lg0),
                    jnp.where(i16 + 16 == l, tot, lg1),
                    jnp.where(i16 + 32 == l, tot, lg2))

        ninf = jnp.full((16,), -jnp.inf, jnp.float32)
        lg0, lg1, lg2 = lax.fori_loop(0, _L, lbody, (ninf, ninf, ninf))

        v0 = lg0 + g_v[pl.ds(0, 16)]
        v1 = lg1 + g_v[pl.ds(16, 16)]
        v2 = lg2 + g_v[pl.ds(32, 16)]
        mx = jnp.maximum(jnp.maximum(jnp.max(v0), jnp.max(v1)), jnp.max(v2))
        big = jnp.int32(1 << 20)
        barg = jnp.minimum(
            jnp.minimum(jnp.min(jnp.where(v0 >= mx, i16, big)),
                        jnp.min(jnp.where(v1 >= mx, i16 + 16, big))),
            jnp.min(jnp.where(v2 >= mx, i16 + 32, big)))

        for k in range(_D // 16):
            out1_v[pl.ds(16 * k, 16)] = seg_v[barg, pl.ds(16 * k, 16)]
        pltpu.sync_copy(out1_v, out1_ref.at[pl.ds(n * _D, _D)])
        rb = barg * (_B * _T) + c * _T + img
        pub_v[...] = jnp.zeros((16,), jnp.int32) + rb
        pltpu.sync_copy(pub_v, shared.at[pl.ds(16 * s, 16)])

    plsc.subcore_barrier()

    @pl.when(s < _NS_C)
    def _spans():
        pltpu.sync_copy(shared, base_v)
        bufs = (stage_v, seg_v.at[pl.ds(0, _RPW)])
        idxb = (idx_v, idx2_v)
        gsem = (sem, sem2)
        wsem = (sem3, sem4)

        def fill_idx(t, b):
            rb = base_v[pl.ds(16 * t, 16)][0]
            for k in range(_RPW // 16):
                idxb[b][pl.ds(16 * k, 16)] = rb + s * _RPW + i16 + 16 * k

        def start_write(t, b):
            dst0 = (c * 3 + t) * _P + s * _RPW
            return pltpu.async_copy(
                bufs[b], out2_ref.at[pl.ds(dst0, _RPW)], wsem[b])

        fill_idx(0, 0)
        gh = [pltpu.async_copy(hs_ref.at[idxb[0]], bufs[0], gsem[0]), None]
        wh = [None, None]
        for t in range(3):
            b = t % 2
            gh[b].wait()
            wh[b] = start_write(t, b)
            if t + 1 < 3:
                nb = 1 - b
                if t >= 1:
                    wh[nb].wait()
                fill_idx(t + 1, nb)
                gh[nb] = pltpu.async_copy(
                    hs_ref.at[idxb[nb]], bufs[nb], gsem[nb])
        wh[0].wait()
        wh[1].wait()


def kernel(hidden_states, W, input_ids, seg_token_mask, num_patches):
    del num_patches
    L, B, T, D = hidden_states.shape
    hs_flat = hidden_states.reshape(L * B * T, D)

    it = jnp.arange(T, dtype=jnp.int32)[None, :]
    img_idx = jnp.min(
        jnp.where(input_ids == _IMAGE_TOKEN_INDEX, it, T), axis=1)
    c0 = jnp.min(jnp.where(seg_token_mask, it, T), axis=1)
    c1 = jnp.min(jnp.where(seg_token_mask & (it > c0[:, None]), it, T), axis=1)
    c2 = jnp.min(jnp.where(seg_token_mask & (it > c1[:, None]), it, T), axis=1)
    cols = jnp.stack([c0, c1, c2], axis=1).reshape(_N)
    meta = jnp.concatenate([
        img_idx.astype(jnp.int32),
        cols.astype(jnp.int32),
        jnp.zeros((16 - _B - _N,), jnp.int32),
    ])
    g = jax.random.gumbel(jax.random.key(1), (_N, _L), jnp.float32)
    g_pad = jnp.concatenate(
        [g, jnp.zeros((_N, _LP - _L), jnp.float32)], axis=1).reshape(_N * _LP)

    mesh = plsc.VectorSubcoreMesh(core_axis_name="c", subcore_axis_name="s")
    out1f, out2f = pl.kernel(
        _sc_body,
        out_type=(
            jax.ShapeDtypeStruct((_N * _D,), jnp.float32),
            jax.ShapeDtypeStruct((_N * _P, _D), jnp.float32),
        ),
        mesh=mesh,
        compiler_params=pltpu.CompilerParams(needs_layout_passes=False),
        scratch_types=[
            pltpu.VMEM((16,), jnp.int32),
            pltpu.VMEM((_LP,), jnp.int32),
            pltpu.VMEM((_LP,), jnp.int32),
            pltpu.VMEM((_LP, _D), jnp.float32),
            pltpu.VMEM((_L, _D), jnp.float32),
            pltpu.VMEM((_LP,), jnp.float32),
            pltpu.VMEM((_D,), jnp.float32),
            pltpu.VMEM((16,), jnp.int32),
            pltpu.VMEM((_LP,), jnp.int32),
            pltpu.VMEM_SHARED((_LP,), jnp.int32),
            pltpu.VMEM((_RPW, _D), jnp.float32),
            pltpu.SemaphoreType.DMA,
            pltpu.SemaphoreType.DMA,
            pltpu.SemaphoreType.DMA,
            pltpu.SemaphoreType.DMA,
        ],
    )(hs_flat, W, meta, g_pad)

    out1 = out1f.reshape(_N, _D)
    out2 = out2f.reshape(_N, _P, _D)
    return (out1, out2, out1)
